# Initial kernel scaffold; baseline (speedup 1.0000x reference)
import jax
import jax.numpy as jnp
from jax.experimental import pallas as pl


def kernel(input_ids, emb, pos_emb, gamma, beta, Wp, bp, Wr, br, Wg, Wu, Wd):
    raise NotImplementedError("write your pallas kernel here")


# speedup vs baseline: 2.2679x; 2.2679x over previous
import functools

import jax
import jax.numpy as jnp
from jax import lax
from jax.experimental import pallas as pl
from jax.experimental.pallas import tpu as pltpu
from jax.experimental.pallas import tpu_sc as plsc


def _make_sc_gather(V, D, N):
    info = plsc.get_sparse_core_info()
    NW = info.num_cores * info.num_subcores
    b_per_w = N // NW
    assert N % NW == 0 and b_per_w % 8 == 0 and D % info.num_lanes == 0
    mesh = plsc.VectorSubcoreMesh(core_axis_name="c", subcore_axis_name="s")

    @functools.partial(
        pl.kernel,
        mesh=mesh,
        out_type=jax.ShapeDtypeStruct((N, D), jnp.float32),
        scratch_types=[
            pltpu.VMEM((b_per_w,), jnp.int32),
            pltpu.VMEM((b_per_w, D), jnp.float32),
            pltpu.SemaphoreType.DMA,
        ],
    )
    def gather_k(table_hbm, idx_hbm, out_hbm, idx_v, rows_v, sem):
        wid = lax.axis_index("s") * info.num_cores + lax.axis_index("c")
        base = wid * b_per_w
        pltpu.sync_copy(idx_hbm.at[pl.ds(base, b_per_w)], idx_v)
        pltpu.async_copy(table_hbm.at[idx_v], rows_v, sem).wait()
        pltpu.sync_copy(rows_v, out_hbm.at[pl.ds(base, b_per_w)])

    return gather_k


def _prep_body(B, S, H, R, E,
               tok_ref, pos_ref, g_ref, be_ref, Wp_ref, bp_ref, Wr_ref,
               br_ref, hn_ref, w_ref, i_ref):
    tok = tok_ref[...]
    h = tok + pos_ref[...][None, :, :]
    mu = jnp.mean(h, axis=-1, keepdims=True)
    var = jnp.mean((h - mu) ** 2, axis=-1, keepdims=True)
    hn = (h - mu) * lax.rsqrt(var + 1e-5) * g_ref[...] + be_ref[...]
    hn_ref[...] = hn
    pooled = jnp.mean(hn, axis=1)
    r = lax.dot_general(pooled, Wp_ref[...], (((1,), (1,)), ((), ())),
                        preferred_element_type=jnp.float32) + bp_ref[...]
    logits = lax.dot_general(r, Wr_ref[...], (((1,), (1,)), ((), ())),
                             preferred_element_type=jnp.float32) + br_ref[...]
    m = jnp.max(logits, axis=1, keepdims=True)
    ex = jnp.exp(logits - m)
    p = ex / jnp.sum(ex, axis=1, keepdims=True)
    idx = lax.broadcasted_iota(jnp.int32, (B, E), 1)
    p1 = jnp.max(p, axis=1, keepdims=True)
    i1 = jnp.min(jnp.where(p == p1, idx, E), axis=1, keepdims=True)
    pm = jnp.where(idx == i1, -1.0, p)
    p2 = jnp.max(pm, axis=1, keepdims=True)
    i2 = jnp.min(jnp.where(pm == p2, idx, E), axis=1, keepdims=True)
    denom = p1 + p2 + 1e-8
    w_ref[...] = jnp.concatenate([p1, p2], axis=1) / denom
    i_ref[...] = jnp.concatenate([i1, i2], axis=1)


def _moe_body(ids_ref, wts_ref, hn_ref, wg_ref, wu_ref, wd_ref, out_ref):
    b = pl.program_id(0)
    k = pl.program_id(1)
    x = hn_ref[0]
    g = jnp.dot(x, wg_ref[0], preferred_element_type=jnp.float32)
    u = jnp.dot(x, wu_ref[0], preferred_element_type=jnp.float32)
    sg = g * (1.0 / (1.0 + jnp.exp(-g)))
    a = sg * u
    o = jnp.dot(a, wd_ref[0], preferred_element_type=jnp.float32)
    w = wts_ref[b, k]

    @pl.when(k == 0)
    def _():
        out_ref[...] = (w * o)[None]

    @pl.when(k != 0)
    def _():
        out_ref[...] += (w * o)[None]


def _logits_body(x_ref, emb_ref, out_ref):
    out_ref[...] = lax.dot_general(
        x_ref[...], emb_ref[...], (((1,), (1,)), ((), ())),
        preferred_element_type=jnp.float32)


def kernel(input_ids, emb, pos_emb, gamma, beta, Wp, bp, Wr, br, Wg, Wu, Wd):
    B, S = input_ids.shape
    V, H = emb.shape
    R = Wp.shape[0]
    E, _, I = Wg.shape
    K = 2
    N = B * S

    ids_flat = input_ids.reshape(N).astype(jnp.int32)
    tok = _make_sc_gather(V, H, N)(emb, ids_flat)
    tok3 = tok.reshape(B, S, H)

    hn, wts, eids = pl.pallas_call(
        functools.partial(_prep_body, B, S, H, R, E),
        out_shape=(
            jax.ShapeDtypeStruct((B, S, H), jnp.float32),
            jax.ShapeDtypeStruct((B, K), jnp.float32),
            jax.ShapeDtypeStruct((B, K), jnp.int32),
        ),
    )(tok3, pos_emb, gamma, beta, Wp, bp, Wr, br)

    grid_spec = pltpu.PrefetchScalarGridSpec(
        num_scalar_prefetch=2,
        grid=(B, K),
        in_specs=[
            pl.BlockSpec((1, S, H), lambda b, k, ids, wts: (b, 0, 0)),
            pl.BlockSpec((1, H, I), lambda b, k, ids, wts: (ids[b, k], 0, 0)),
            pl.BlockSpec((1, H, I), lambda b, k, ids, wts: (ids[b, k], 0, 0)),
            pl.BlockSpec((1, I, H), lambda b, k, ids, wts: (ids[b, k], 0, 0)),
        ],
        out_specs=pl.BlockSpec((1, S, H), lambda b, k, ids, wts: (b, 0, 0)),
    )
    combined = pl.pallas_call(
        _moe_body,
        grid_spec=grid_spec,
        out_shape=jax.ShapeDtypeStruct((B, S, H), jnp.float32),
    )(eids, wts, hn, Wg, Wu, Wd)

    VT = 1280
    logits = pl.pallas_call(
        _logits_body,
        grid=(V // VT,),
        in_specs=[
            pl.BlockSpec((N, H), lambda v: (0, 0)),
            pl.BlockSpec((VT, H), lambda v: (v, 0)),
        ],
        out_specs=pl.BlockSpec((N, VT), lambda v: (0, v)),
        out_shape=jax.ShapeDtypeStruct((N, V), jnp.float32),
    )(combined.reshape(N, H), emb)

    return logits.reshape(B, S, V)
